# baseline (speedup 1.0000x reference)
import functools

import jax
import jax.numpy as jnp
from jax.experimental import pallas as pl

_IN = 4
_NC = 16
_HID = 100
_HP = 128
_PAD = -100.0


def _fwd_kernel(xc_ref, w1_ref, w2_ref, out_ref, *, Q):
    xd = jnp.concatenate([xc_ref[0, g].T for g in range(8)], axis=1)
    w1 = w1_ref[...]
    w2 = w2_ref[...]
    ones = jnp.ones((1, Q), jnp.float32)
    pad1 = jnp.full((1, 1), _PAD, jnp.float32)

    phases = []
    for j in range(8):
        rows = []
        for k in range(_IN):
            o = j - 3 + k
            if o >= 0:
                rows.append(xd[o:o + 1, :])
            else:
                rows.append(
                    jnp.concatenate([pad1, xd[o + 8:o + 9, :Q - 1]], axis=1))
        rows.append(ones)
        xt = jnp.concatenate(rows, axis=0)

        ht = jnp.dot(w1, xt, preferred_element_type=jnp.float32)
        ht = jnp.maximum(ht, 0.0)
        lt = jnp.dot(w2, ht, preferred_element_type=jnp.float32)

        m = jnp.max(lt, axis=0, keepdims=True)
        s = jnp.sum(jnp.exp(lt - m), axis=0, keepdims=True)
        phases.append(lt - m - jnp.log(s))

    out128 = jnp.concatenate(phases, axis=0)
    out_ref[0] = out128.T


@functools.partial(jax.jit, static_argnames=("interpret",))
def kernel(input_, W1, b1, W2, b2, interpret=False):
    B, T = input_.shape
    Q = T // 8

    w1a = jnp.zeros((_HP, _IN + 1), jnp.float32)
    w1a = w1a.at[:_HID, :_IN].set(W1.T).at[:_HID, _IN].set(b1)
    w1a = w1a.at[_HP - 1, _IN].set(1.0)
    w2a = jnp.zeros((_NC, _HP), jnp.float32)
    w2a = w2a.at[:, :_HID].set(W2.T).at[:, _HP - 1].set(b2)

    x4 = input_.reshape(B, 8, 128, 8)

    out = pl.pallas_call(
        functools.partial(_fwd_kernel, Q=Q),
        grid=(B,),
        in_specs=[
            pl.BlockSpec((1, 8, 128, 8), lambda b: (b, 0, 0, 0)),
            pl.BlockSpec((_HP, _IN + 1), lambda b: (0, 0)),
            pl.BlockSpec((_NC, _HP), lambda b: (0, 0)),
        ],
        out_specs=pl.BlockSpec((1, Q, _HP), lambda b: (b, 0, 0)),
        out_shape=jax.ShapeDtypeStruct((B, Q, _HP), jnp.float32),
        interpret=interpret,
    )(x4, w1a, w2a)
    return out.reshape(B, T, _NC)


# speedup vs baseline: 2.3934x; 2.3934x over previous
import functools

import jax
import jax.numpy as jnp
from jax.experimental import pallas as pl

_IN = 4
_NC = 16
_HID = 100
_HP = 128
_PAD = -100.0


def _fwd_kernel(xc_ref, w1_ref, w2_ref, out_ref, *, T):
    xc = xc_ref[0]
    pad = jnp.full((1, 3), _PAD, jnp.float32)
    xe = jnp.concatenate([pad, xc], axis=1)

    xt = jnp.concatenate(
        [xe[:, 0:T], xe[:, 1:T + 1], xe[:, 2:T + 2], xe[:, 3:T + 3],
         jnp.ones((1, T), jnp.float32)],
        axis=0,
    )

    ht = jnp.dot(w1_ref[...], xt, preferred_element_type=jnp.float32)
    ht = jnp.maximum(ht, 0.0)

    lt = jnp.dot(w2_ref[...], ht, preferred_element_type=jnp.float32)

    m = jnp.max(lt, axis=0, keepdims=True)
    s = jnp.sum(jnp.exp(lt - m), axis=0, keepdims=True)
    out_ref[0] = lt - m - jnp.log(s)


@functools.partial(jax.jit, static_argnames=("interpret",))
def kernel(input_, W1, b1, W2, b2, interpret=False):
    B, T = input_.shape

    w1a = jnp.zeros((_HP, _IN + 1), jnp.float32)
    w1a = w1a.at[:_HID, :_IN].set(W1.T).at[:_HID, _IN].set(b1)
    w1a = w1a.at[_HP - 1, _IN].set(1.0)
    w2a = jnp.zeros((_NC, _HP), jnp.float32)
    w2a = w2a.at[:, :_HID].set(W2.T).at[:, _HP - 1].set(b2)

    x3 = input_.reshape(B, 1, T)

    out = pl.pallas_call(
        functools.partial(_fwd_kernel, T=T),
        grid=(B,),
        in_specs=[
            pl.BlockSpec((1, 1, T), lambda b: (b, 0, 0)),
            pl.BlockSpec((_HP, _IN + 1), lambda b: (0, 0)),
            pl.BlockSpec((_NC, _HP), lambda b: (0, 0)),
        ],
        out_specs=pl.BlockSpec((1, _NC, T), lambda b: (b, 0, 0)),
        out_shape=jax.ShapeDtypeStruct((B, _NC, T), jnp.float32),
        interpret=interpret,
    )(x3, w1a, w2a)
    return out.transpose(0, 2, 1)
